# Initial kernel scaffold; baseline (speedup 1.0000x reference)
import jax
import jax.numpy as jnp
from jax.experimental import pallas as pl


def kernel(x, running_mean, running_var):
    raise NotImplementedError("write your pallas kernel here")


# speedup vs baseline: 1.2647x; 1.2647x over previous
import jax
import jax.numpy as jnp
from jax.experimental import pallas as pl
from jax.experimental.pallas import tpu as pltpu

_EPS = 1e-5
_SMOOTHING = 0.9
_TILE_F = 256
_VMEM_LIMIT = 64 * 1024 * 1024


def _cbn_kernel(x_ref, rm_ref, rv_ref, y_ref, nrm_ref, nrv_ref):
    x = x_ref[...]
    n = jnp.float32(x.shape[0])
    s1 = jnp.sum(x, axis=0, keepdims=True)
    s2 = jnp.sum(x * x, axis=0, keepdims=True)
    mean = s1 * (1.0 / n)
    var = jnp.maximum(s2 * (1.0 / n) - mean * mean, 0.0)
    scale = jax.lax.rsqrt(var + _EPS)
    shift = -mean * scale
    y_ref[...] = x * scale + shift
    nrm_ref[...] = _SMOOTHING * rm_ref[...] + (1.0 - _SMOOTHING) * mean
    nrv_ref[...] = _SMOOTHING * rv_ref[...] + (1.0 - _SMOOTHING) * var


def kernel(x, running_mean, running_var):
    n, f = x.shape
    tf = _TILE_F if f % _TILE_F == 0 else f
    rm = running_mean.reshape(1, f)
    rv = running_var.reshape(1, f)
    x_spec = pl.BlockSpec((n, tf), lambda j: (0, j))
    r_spec = pl.BlockSpec((1, tf), lambda j: (0, j))
    y, nrm, nrv = pl.pallas_call(
        _cbn_kernel,
        out_shape=(
            jax.ShapeDtypeStruct((n, f), x.dtype),
            jax.ShapeDtypeStruct((1, f), running_mean.dtype),
            jax.ShapeDtypeStruct((1, f), running_var.dtype),
        ),
        grid=(f // tf,),
        in_specs=[x_spec, r_spec, r_spec],
        out_specs=(x_spec, r_spec, r_spec),
        compiler_params=pltpu.CompilerParams(
            dimension_semantics=("parallel",),
            vmem_limit_bytes=_VMEM_LIMIT),
    )(x, rm, rv)
    return y, nrm.reshape(f), nrv.reshape(f)


# speedup vs baseline: 1.3084x; 1.0345x over previous
import jax
import jax.numpy as jnp
from jax.experimental import pallas as pl
from jax.experimental.pallas import tpu as pltpu

_EPS = 1e-5
_SMOOTHING = 0.9
_TILE_F = 384
_VMEM_LIMIT = 64 * 1024 * 1024


def _cbn_kernel(x_ref, rm_ref, rv_ref, y_ref, nrm_ref, nrv_ref):
    x = x_ref[...]
    n = jnp.float32(x.shape[0])
    s1 = jnp.sum(x, axis=0, keepdims=True)
    s2 = jnp.sum(x * x, axis=0, keepdims=True)
    mean = s1 * (1.0 / n)
    var = jnp.maximum(s2 * (1.0 / n) - mean * mean, 0.0)
    scale = jax.lax.rsqrt(var + _EPS)
    shift = -mean * scale
    y_ref[...] = x * scale + shift
    nrm_ref[...] = _SMOOTHING * rm_ref[...] + (1.0 - _SMOOTHING) * mean
    nrv_ref[...] = _SMOOTHING * rv_ref[...] + (1.0 - _SMOOTHING) * var


def kernel(x, running_mean, running_var):
    n, f = x.shape
    tf = min(_TILE_F, f)
    rm = running_mean.reshape(1, f)
    rv = running_var.reshape(1, f)
    x_spec = pl.BlockSpec((n, tf), lambda j: (0, j))
    r_spec = pl.BlockSpec((1, tf), lambda j: (0, j))
    y, nrm, nrv = pl.pallas_call(
        _cbn_kernel,
        out_shape=(
            jax.ShapeDtypeStruct((n, f), x.dtype),
            jax.ShapeDtypeStruct((1, f), running_mean.dtype),
            jax.ShapeDtypeStruct((1, f), running_var.dtype),
        ),
        grid=(pl.cdiv(f, tf),),
        in_specs=[x_spec, r_spec, r_spec],
        out_specs=(x_spec, r_spec, r_spec),
        compiler_params=pltpu.CompilerParams(
            dimension_semantics=("parallel",),
            vmem_limit_bytes=_VMEM_LIMIT),
    )(x, rm, rv)
    return y, nrm.reshape(f), nrv.reshape(f)
